# baseline (speedup 1.0000x reference)
import dataclasses

import jax
import jax.numpy as jnp
from jax import lax
from jax.experimental import pallas as pl
from jax.experimental.pallas import tpu as pltpu
from jax.experimental.pallas import tpu_sc as plsc

N = 10000
D = 128
E = 320000
CHUNK = 128
NCORES = 2
NSUB = 16
NW = NCORES * NSUB
NBUF = 2
IDXL = 4
NCHUNKS = 2560
EP = NCHUNKS * CHUNK
CPW = NCHUNKS // NW
NP = 10240
RPS = NP // NSUB
BLK = 1024
NB = NP // BLK


def _build_sc_aggregate(with_cnt):
    mesh = plsc.VectorSubcoreMesh(core_axis_name="c", subcore_axis_name="s")
    out_type = [jax.ShapeDtypeStruct((NCORES * NP, D), jnp.float32)]
    scratch_types = [
        pltpu.VMEM((CHUNK,), jnp.int32),
        pltpu.VMEM((CHUNK,), jnp.int32),
        pltpu.VMEM((CHUNK,), jnp.int32),
        pltpu.VMEM((CHUNK,), jnp.int32),
        pltpu.VMEM((CHUNK, D), jnp.float32),
        pltpu.VMEM_SHARED((NP, D), jnp.float32),
        pltpu.SemaphoreType.DMA((2,)),
        pltpu.SemaphoreType.DMA,
    ]
    if with_cnt:
        out_type.append(jax.ShapeDtypeStruct((NW, NP), jnp.float32))
        scratch_types.append(pltpu.VMEM((NP,), jnp.float32))

    def body(table_hbm, e1_hbm, zacc_hbm, *rest):
        if with_cnt:
            (out_hbm, outcnt_hbm, src0_v, dst0_v, src1_v, dst1_v, rows_v,
             acc_sh, isem, gsem, cnt_v) = rest
        else:
            (out_hbm, src0_v, dst0_v, src1_v, dst1_v, rows_v, acc_sh, isem,
             gsem) = rest
            outcnt_hbm = cnt_v = None

        cid = lax.axis_index("c")
        sid = lax.axis_index("s")
        wid = sid * NCORES + cid
        base = wid * CPW
        slots = [(src0_v, dst0_v), (src1_v, dst1_v)]

        def _idx_start(j, b):
            sv, dv = slots[b]
            off = (base + j) * 2 * CHUNK
            pltpu.make_async_copy(e1_hbm.at[pl.ds(off, CHUNK)], sv,
                                  isem.at[b]).start()
            pltpu.make_async_copy(e1_hbm.at[pl.ds(off + CHUNK, CHUNK)], dv,
                                  isem.at[b]).start()

        def _idx_wait(b):
            sv, dv = slots[b]
            pltpu.make_async_copy(e1_hbm.at[pl.ds(0, CHUNK)], sv,
                                  isem.at[b]).wait()
            pltpu.make_async_copy(e1_hbm.at[pl.ds(0, CHUNK)], dv,
                                  isem.at[b]).wait()

        _idx_start(0, 0)
        pltpu.sync_copy(zacc_hbm.at[pl.ds(sid * RPS, RPS)],
                        acc_sh.at[pl.ds(sid * RPS, RPS)])
        if with_cnt:
            @pl.loop(0, NP, step=16)
            def _(i):
                cnt_v[pl.ds(i, 16)] = jnp.zeros((16,), jnp.float32)
        plsc.subcore_barrier()

        @pl.loop(0, CPW, step=2)
        def _(jj):
            for b in range(2):
                j = jj + b
                sv, dv = slots[b]
                _idx_wait(b)
                gather = pltpu.make_async_copy(table_hbm.at[sv], rows_v, gsem)
                gather.start()

                @pl.when(j + 1 < CPW)
                def _():
                    _idx_start(j + 1, 1 - b)

                gather.wait()
                pltpu.sync_copy(rows_v, acc_sh.at[dv], add=True)
                if with_cnt:
                    for k in range(CHUNK // 16):
                        plsc.addupdate_scatter(
                            cnt_v, [dv[pl.ds(k * 16, 16)]],
                            jnp.ones((16,), jnp.float32))

        plsc.subcore_barrier()

        pltpu.sync_copy(acc_sh.at[pl.ds(sid * RPS, RPS)],
                        out_hbm.at[pl.ds(cid * NP + sid * RPS, RPS)])
        if with_cnt:
            pltpu.sync_copy(cnt_v, outcnt_hbm.at[wid])

    cp = pltpu.CompilerParams()
    if with_cnt and "needs_layout_passes" in pltpu.CompilerParams.__dataclass_fields__:
        cp = dataclasses.replace(cp, needs_layout_passes=False)
    return pl.kernel(body, out_type=out_type, mesh=mesh,
                     scratch_types=scratch_types, compiler_params=cp)


_sc_agg_cnt = _build_sc_aggregate(with_cnt=True)
_sc_agg = _build_sc_aggregate(with_cnt=False)


def _tc_layer(aggflat, cntp, xp, W_l, W_r, b, relu):
    def body(agg0_ref, agg1_ref, cnt_ref, x_ref, wl_ref, wr_ref, b_ref, o_ref):
        cnt = jnp.sum(cnt_ref[...], axis=0)[:, None]
        inv = 1.0 / jnp.maximum(cnt, 1.0)
        mean = (agg0_ref[...] + agg1_ref[...]) * inv
        y = (jnp.dot(mean, wl_ref[...], preferred_element_type=jnp.float32)
             + jnp.dot(x_ref[...], wr_ref[...], preferred_element_type=jnp.float32)
             + b_ref[...])
        if relu:
            y = jnp.maximum(y, 0.0)
        o_ref[...] = y

    return pl.pallas_call(
        body,
        grid=(NB,),
        in_specs=[
            pl.BlockSpec((BLK, D), lambda i: (i, 0)),
            pl.BlockSpec((BLK, D), lambda i: (NB + i, 0)),
            pl.BlockSpec((NW, BLK), lambda i: (0, i)),
            pl.BlockSpec((BLK, D), lambda i: (i, 0)),
            pl.BlockSpec((D, D), lambda i: (0, 0)),
            pl.BlockSpec((D, D), lambda i: (0, 0)),
            pl.BlockSpec((1, D), lambda i: (0, 0)),
        ],
        out_specs=pl.BlockSpec((BLK, D), lambda i: (i, 0)),
        out_shape=jax.ShapeDtypeStruct((NP, D), jnp.float32),
    )(aggflat, aggflat, cntp, xp, W_l, W_r, b.reshape(1, D))


def kernel(x, edge_index, W1_l, W1_r, b1, W2_l, W2_r, b2):
    src = edge_index[0].astype(jnp.int32)
    dst = edge_index[1].astype(jnp.int32)
    pad = EP - E
    src_p = jnp.concatenate([src, jnp.zeros((pad,), jnp.int32)])
    dst_p = jnp.concatenate([dst, jnp.full((pad,), N, jnp.int32)])
    e1 = jnp.stack([src_p.reshape(NCHUNKS, CHUNK),
                    dst_p.reshape(NCHUNKS, CHUNK)], axis=1).reshape(-1)

    x_pad = jnp.pad(x, ((0, NP - N), (0, 0)))
    zacc = jnp.zeros((NP, D), jnp.float32)

    agg1, cntp = _sc_agg_cnt(x_pad, e1, zacc)
    h = _tc_layer(agg1, cntp, x_pad, W1_l, W1_r, b1, relu=True)
    (agg2,) = _sc_agg(h, e1, zacc)
    out = _tc_layer(agg2, cntp, h, W2_l, W2_r, b2, relu=False)
    return out[:N]


# speedup vs baseline: 1.3893x; 1.3893x over previous
import dataclasses

import jax
import jax.numpy as jnp
from jax import lax
from jax.experimental import pallas as pl
from jax.experimental.pallas import tpu as pltpu
from jax.experimental.pallas import tpu_sc as plsc

N = 10000
D = 128
E = 320000
CHUNK = 128
NCORES = 2
NSUB = 16
NW = NCORES * NSUB
NCHUNKS = 2528
EP = NCHUNKS * CHUNK
CPW = NCHUNKS // NW
NP = 10240
RPS = NP // NSUB
BLK = 1024
NB = NP // BLK


def _build_sc_aggregate(with_cnt):
    mesh = plsc.VectorSubcoreMesh(core_axis_name="c", subcore_axis_name="s")
    out_type = [jax.ShapeDtypeStruct((NCORES * NP, D), jnp.float32)]
    scratch_types = [
        pltpu.VMEM((CHUNK,), jnp.int32),
        pltpu.VMEM((CHUNK,), jnp.int32),
        pltpu.VMEM((CHUNK, D), jnp.float32),
        pltpu.VMEM_SHARED((NP, D), jnp.float32),
        pltpu.SemaphoreType.DMA,
        pltpu.SemaphoreType.DMA,
    ]
    if with_cnt:
        out_type.append(jax.ShapeDtypeStruct((NW, NP), jnp.float32))
        scratch_types.append(pltpu.VMEM((NP,), jnp.float32))

    def body(table_hbm, src_hbm, dst_hbm, zacc_hbm, *rest):
        if with_cnt:
            (out_hbm, outcnt_hbm, src_v, dst_v, rows_v, acc_sh, sem, isem,
             cnt_v) = rest
        else:
            out_hbm, src_v, dst_v, rows_v, acc_sh, sem, isem = rest
            outcnt_hbm = cnt_v = None

        cid = lax.axis_index("c")
        sid = lax.axis_index("s")
        wid = sid * NCORES + cid

        pltpu.sync_copy(zacc_hbm.at[pl.ds(sid * RPS, RPS)],
                        acc_sh.at[pl.ds(sid * RPS, RPS)])
        if with_cnt:
            @pl.loop(0, NP, step=16)
            def _(i):
                cnt_v[pl.ds(i, 16)] = jnp.zeros((16,), jnp.float32)
        plsc.subcore_barrier()

        base = wid * CPW

        @pl.loop(0, CPW)
        def _(j):
            off = (base + j) * CHUNK
            ca = pltpu.make_async_copy(src_hbm.at[pl.ds(off, CHUNK)], src_v,
                                       isem)
            cb = pltpu.make_async_copy(dst_hbm.at[pl.ds(off, CHUNK)], dst_v,
                                       isem)
            ca.start()
            cb.start()
            ca.wait()
            cb.wait()
            gather = pltpu.make_async_copy(table_hbm.at[src_v], rows_v, sem)
            gather.start()
            if with_cnt:
                for k in range(CHUNK // 16):
                    plsc.addupdate_scatter(
                        cnt_v, [dst_v[pl.ds(k * 16, 16)]],
                        jnp.ones((16,), jnp.float32))
            gather.wait()
            pltpu.sync_copy(rows_v, acc_sh.at[dst_v], add=True)

        plsc.subcore_barrier()

        pltpu.sync_copy(acc_sh.at[pl.ds(sid * RPS, RPS)],
                        out_hbm.at[pl.ds(cid * NP + sid * RPS, RPS)])
        if with_cnt:
            pltpu.sync_copy(cnt_v, outcnt_hbm.at[wid])

    cp = pltpu.CompilerParams()
    if with_cnt and "needs_layout_passes" in pltpu.CompilerParams.__dataclass_fields__:
        cp = dataclasses.replace(cp, needs_layout_passes=False)
    return pl.kernel(body, out_type=out_type, mesh=mesh,
                     scratch_types=scratch_types, compiler_params=cp)


_sc_agg_cnt = _build_sc_aggregate(with_cnt=True)
_sc_agg = _build_sc_aggregate(with_cnt=False)


def _tc_layer(aggflat, cntp, xp, W_l, W_r, b, relu):
    def body(agg0_ref, agg1_ref, cnt_ref, x_ref, wl_ref, wr_ref, b_ref, o_ref):
        cnt = jnp.sum(cnt_ref[...], axis=0)[:, None]
        inv = 1.0 / jnp.maximum(cnt, 1.0)
        mean = (agg0_ref[...] + agg1_ref[...]) * inv
        y = (jnp.dot(mean, wl_ref[...], preferred_element_type=jnp.float32)
             + jnp.dot(x_ref[...], wr_ref[...], preferred_element_type=jnp.float32)
             + b_ref[...])
        if relu:
            y = jnp.maximum(y, 0.0)
        o_ref[...] = y

    return pl.pallas_call(
        body,
        grid=(NB,),
        in_specs=[
            pl.BlockSpec((BLK, D), lambda i: (i, 0)),
            pl.BlockSpec((BLK, D), lambda i: (NB + i, 0)),
            pl.BlockSpec((NW, BLK), lambda i: (0, i)),
            pl.BlockSpec((BLK, D), lambda i: (i, 0)),
            pl.BlockSpec((D, D), lambda i: (0, 0)),
            pl.BlockSpec((D, D), lambda i: (0, 0)),
            pl.BlockSpec((1, D), lambda i: (0, 0)),
        ],
        out_specs=pl.BlockSpec((BLK, D), lambda i: (i, 0)),
        out_shape=jax.ShapeDtypeStruct((NP, D), jnp.float32),
    )(aggflat, aggflat, cntp, xp, W_l, W_r, b.reshape(1, D))


def kernel(x, edge_index, W1_l, W1_r, b1, W2_l, W2_r, b2):
    src = edge_index[0].astype(jnp.int32)
    dst = edge_index[1].astype(jnp.int32)
    pad = EP - E
    src_p = jnp.concatenate([src, jnp.zeros((pad,), jnp.int32)])
    dst_p = jnp.concatenate([dst, jnp.full((pad,), N, jnp.int32)])

    x_pad = jnp.pad(x, ((0, NP - N), (0, 0)))
    zacc = jnp.zeros((NP, D), jnp.float32)

    agg1, cntp = _sc_agg_cnt(x_pad, src_p, dst_p, zacc)
    h = _tc_layer(agg1, cntp, x_pad, W1_l, W1_r, b1, relu=True)
    (agg2,) = _sc_agg(h, src_p, dst_p, zacc)
    out = _tc_layer(agg2, cntp, h, W2_l, W2_r, b2, relu=False)
    return out[:N]
